# baseline (speedup 1.0000x reference)
import functools

import jax
import jax.numpy as jnp
from jax import lax
from jax.experimental import pallas as pl
from jax.experimental.pallas import tpu as pltpu

N = 50000
E = 800000
D_IN = 58
H = 64
G = 256

BN = 2000


def _inproj_body(x_ref, w_ref, b_ref, o_ref):
    o_ref[...] = jax.nn.relu(
        jnp.dot(x_ref[...], w_ref[...], preferred_element_type=jnp.float32)
        + b_ref[...]
    )


def _input_proj(x, W_in, b_in):
    return pl.pallas_call(
        _inproj_body,
        grid=(N // BN,),
        in_specs=[
            pl.BlockSpec((BN, D_IN), lambda i: (i, 0)),
            pl.BlockSpec((D_IN, H), lambda i: (0, 0)),
            pl.BlockSpec((1, H), lambda i: (0, 0)),
        ],
        out_specs=pl.BlockSpec((BN, H), lambda i: (i, 0)),
        out_shape=jax.ShapeDtypeStruct((N, H), jnp.float32),
    )(x, W_in, b_in.reshape(1, H))


def _layer_body(h_ref, s_ref, c_ref, wt_ref, wb_ref, b_ref, o_ref):
    inv = 1.0 / jnp.maximum(c_ref[...], 1.0)
    agg = s_ref[...] * inv
    acc = jnp.dot(h_ref[...], wt_ref[...], preferred_element_type=jnp.float32)
    acc += jnp.dot(agg, wb_ref[...], preferred_element_type=jnp.float32)
    o_ref[...] = jax.nn.relu(acc + b_ref[...])


def _layer_dense(h, summed, cnt, W, b):
    W_top = W[:H]
    W_bot = W[H:]
    return pl.pallas_call(
        _layer_body,
        grid=(N // BN,),
        in_specs=[
            pl.BlockSpec((BN, H), lambda i: (i, 0)),
            pl.BlockSpec((BN, H), lambda i: (i, 0)),
            pl.BlockSpec((BN, 1), lambda i: (i, 0)),
            pl.BlockSpec((H, H), lambda i: (0, 0)),
            pl.BlockSpec((H, H), lambda i: (0, 0)),
            pl.BlockSpec((1, H), lambda i: (0, 0)),
        ],
        out_specs=pl.BlockSpec((BN, H), lambda i: (i, 0)),
        out_shape=jax.ShapeDtypeStruct((N, H), jnp.float32),
    )(h, summed, cnt, W_top, W_bot, b.reshape(1, H))


def _heads_body(gs_ref, gc_ref, w_ref, b_ref, o_ref):
    hg = gs_ref[...] / jnp.maximum(gc_ref[...], 1.0)
    z = jnp.dot(hg, w_ref[...], preferred_element_type=jnp.float32) + b_ref[...]
    qed = jax.nn.sigmoid(z[:, 1:2])
    o_ref[...] = jnp.concatenate([z[:, 0:1], qed, z[:, 2:4]], axis=1)


def _heads(gsum, gcnt, W_heads, b_heads):
    return pl.pallas_call(
        _heads_body,
        out_shape=jax.ShapeDtypeStruct((G, 4), jnp.float32),
    )(gsum, gcnt, W_heads, b_heads.reshape(1, 4))


def kernel(x, edge_index, batch, W_in, b_in, W1, b1, W2, b2,
           W_aff, b_aff, W_qed, b_qed, W_sa, b_sa, W_tpsa, b_tpsa):
    src = edge_index[0]
    dst = edge_index[1]

    h = _input_proj(x, W_in, b_in)

    cnt = jax.ops.segment_sum(jnp.ones((E, 1), jnp.float32), dst,
                              num_segments=N)
    s1 = jax.ops.segment_sum(h[src], dst, num_segments=N)
    h = _layer_dense(h, s1, cnt, W1, b1)
    s2 = jax.ops.segment_sum(h[src], dst, num_segments=N)
    h = _layer_dense(h, s2, cnt, W2, b2)

    gsum = jax.ops.segment_sum(h, batch, num_segments=G)
    gcnt = jax.ops.segment_sum(jnp.ones((N, 1), jnp.float32), batch,
                               num_segments=G)

    W_heads = jnp.concatenate([W_aff, W_qed, W_sa, W_tpsa], axis=1)
    b_heads = jnp.concatenate([b_aff, b_qed, b_sa, b_tpsa])
    out = _heads(gsum, gcnt, W_heads, b_heads)
    return (out[:, 0], out[:, 1], out[:, 2], out[:, 3])


# speedup vs baseline: 9.0636x; 9.0636x over previous
import functools

import jax
import jax.numpy as jnp
from jax import lax
from jax.experimental import pallas as pl
from jax.experimental.pallas import tpu as pltpu
from jax.experimental.pallas import tpu_sc as plsc

N = 50000
E = 800000
D_IN = 58
H = 64
HH = 32
G = 256

BN = 2000
CH = 128
NROW = E // CH
NTILE = 16
NBUF = 4
RPT = 3128

_BASE = NROW // NTILE
_EXTRA = NROW - _BASE * NTILE
_MCH = _BASE + 1


def _inproj_body(x_ref, w_ref, b_ref, o_ref):
    acc = jax.nn.relu(
        jnp.dot(x_ref[...], w_ref[...], preferred_element_type=jnp.float32)
        + b_ref[...]
    )
    o_ref[0] = acc[:, :HH]
    o_ref[1] = acc[:, HH:]


def _input_proj(x, W_in, b_in):
    return pl.pallas_call(
        _inproj_body,
        grid=(N // BN,),
        in_specs=[
            pl.BlockSpec((BN, D_IN), lambda i: (i, 0)),
            pl.BlockSpec((D_IN, H), lambda i: (0, 0)),
            pl.BlockSpec((1, H), lambda i: (0, 0)),
        ],
        out_specs=pl.BlockSpec((2, BN, HH), lambda i: (0, i, 0)),
        out_shape=jax.ShapeDtypeStruct((2, N, HH), jnp.float32),
    )(x, W_in, b_in.reshape(1, H))


def _per_tile_slice(s, fn):
    pl.when(s < NTILE - 1)(lambda: fn(s * RPT, RPT))
    pl.when(s == NTILE - 1)(lambda: fn((NTILE - 1) * RPT,
                                       N - (NTILE - 1) * RPT))


def _make_sc_agg():
    mesh = plsc.VectorSubcoreMesh(core_axis_name="c", subcore_axis_name="s")

    scratch = dict(
        sidx=pltpu.VMEM((NBUF, CH), jnp.int32),
        didx=pltpu.VMEM((NBUF, CH), jnp.int32),
        rows=pltpu.VMEM((NBUF, CH, HH), jnp.float32),
        accum=pltpu.VMEM_SHARED((N, HH), jnp.float32),
        isem_s=pltpu.SemaphoreType.DMA((NBUF,)),
        isem_d=pltpu.SemaphoreType.DMA((NBUF,)),
        gsem=pltpu.SemaphoreType.DMA((NBUF,)),
    )

    def body(h_hbm, src2d, dst2d, z32_hbm, s_out,
             sidx, didx, rows, accum, isem_s, isem_d, gsem):
        c = lax.axis_index("c")
        s = lax.axis_index("s")
        table = h_hbm.at[c]

        _per_tile_slice(s, lambda r0, ln: pltpu.sync_copy(
            z32_hbm.at[pl.ds(r0, ln)], accum.at[pl.ds(r0, ln)]))
        plsc.subcore_barrier()

        nch = _BASE + jnp.where(s < _EXTRA, 1, 0)
        start = s * _BASE + jnp.minimum(s, _EXTRA)

        def issue_idx(k):
            b = lax.rem(k, NBUF)
            row = start + k
            pltpu.async_copy(src2d.at[row], sidx.at[b], isem_s.at[b])
            pltpu.async_copy(dst2d.at[row], didx.at[b], isem_d.at[b])

        def issue_gather(k):
            b = lax.rem(k, NBUF)
            row = start + k
            pltpu.make_async_copy(src2d.at[row], sidx.at[b],
                                  isem_s.at[b]).wait()
            pltpu.async_copy(table.at[sidx.at[b]], rows.at[b], gsem.at[b])

        def consume(k):
            b = lax.rem(k, NBUF)
            row = start + k
            pltpu.make_async_copy(dst2d.at[row], didx.at[b],
                                  isem_d.at[b]).wait()
            pltpu.make_async_copy(table.at[sidx.at[b]], rows.at[b],
                                  gsem.at[b]).wait()
            pltpu.sync_copy(rows.at[b], accum.at[didx.at[b]], add=True)

        def step(k, _):
            pl.when(k < nch)(lambda: issue_idx(k))
            pl.when((k >= 1) & (k - 1 < nch))(lambda: issue_gather(k - 1))
            pl.when((k >= 2) & (k - 2 < nch))(lambda: consume(k - 2))
            return _

        lax.fori_loop(0, _MCH + 2, step, None)

        plsc.subcore_barrier()

        _per_tile_slice(s, lambda r0, ln: pltpu.sync_copy(
            accum.at[pl.ds(r0, ln)], s_out.at[c, pl.ds(r0, ln)]))

    return pl.kernel(
        body,
        out_type=jax.ShapeDtypeStruct((2, N, HH), jnp.float32),
        mesh=mesh,
        scratch_types=scratch,
        compiler_params=pltpu.CompilerParams(use_tc_tiling_on_sc=False),
    )


SLAB = 10
_NSLAB = NROW // SLAB
_NW = 2 * NTILE
_SBASE = _NSLAB // _NW
_SEXTRA = _NSLAB - _SBASE * _NW
_MSL = _SBASE + 1


def _make_sc_counts():
    mesh = plsc.VectorSubcoreMesh(core_axis_name="c", subcore_axis_name="s")

    scratch = dict(
        slab=pltpu.VMEM((NBUF, SLAB, CH), jnp.int32),
        ones_v=pltpu.VMEM((CH, 8), jnp.float32),
        cnt_sh=pltpu.VMEM_SHARED((N, 8), jnp.float32),
        dsem=pltpu.SemaphoreType.DMA((NBUF,)),
    )

    def body(dst2d, z8_hbm, ones_hbm, c_out,
             slab, ones_v, cnt_sh, dsem):
        c = lax.axis_index("c")
        s = lax.axis_index("s")
        w = c * NTILE + s

        _per_tile_slice(s, lambda r0, ln: pltpu.sync_copy(
            z8_hbm.at[pl.ds(r0, ln)], cnt_sh.at[pl.ds(r0, ln)]))
        pltpu.sync_copy(ones_hbm, ones_v)
        plsc.subcore_barrier()

        nsl = _SBASE + jnp.where(w < _SEXTRA, 1, 0)
        start = w * _SBASE + jnp.minimum(w, _SEXTRA)

        def issue(k):
            b = lax.rem(k, NBUF)
            row0 = (start + k) * SLAB
            pltpu.async_copy(dst2d.at[pl.ds(row0, SLAB)], slab.at[b],
                             dsem.at[b])

        def consume(k):
            b = lax.rem(k, NBUF)
            row0 = (start + k) * SLAB
            pltpu.make_async_copy(dst2d.at[pl.ds(row0, SLAB)], slab.at[b],
                                  dsem.at[b]).wait()
            for j in range(SLAB):
                pltpu.sync_copy(ones_v, cnt_sh.at[slab.at[b, j]], add=True)

        def step(k, _):
            pl.when(k < nsl)(lambda: issue(k))
            pl.when((k >= 1) & (k - 1 < nsl))(lambda: consume(k - 1))
            return _

        lax.fori_loop(0, _MSL + 1, step, None)

        plsc.subcore_barrier()

        _per_tile_slice(s, lambda r0, ln: pltpu.sync_copy(
            cnt_sh.at[pl.ds(r0, ln)], c_out.at[c, pl.ds(r0, ln)]))

    return pl.kernel(
        body,
        out_type=jax.ShapeDtypeStruct((2, N, 8), jnp.float32),
        mesh=mesh,
        scratch_types=scratch,
        compiler_params=pltpu.CompilerParams(use_tc_tiling_on_sc=False),
    )


def _layer_body(h_ref, s_ref, c_ref, w_ref, b_ref, o_ref):
    cnt = c_ref[0][:, 0:1] + c_ref[1][:, 0:1]
    inv = 1.0 / jnp.maximum(cnt, 1.0)
    h_full = jnp.concatenate([h_ref[0], h_ref[1]], axis=1)
    agg = jnp.concatenate([s_ref[0], s_ref[1]], axis=1) * inv
    acc = jnp.dot(h_full, w_ref[:H], preferred_element_type=jnp.float32)
    acc += jnp.dot(agg, w_ref[H:], preferred_element_type=jnp.float32)
    acc = jax.nn.relu(acc + b_ref[...])
    o_ref[0] = acc[:, :HH]
    o_ref[1] = acc[:, HH:]


def _layer_dense(h, summed, cnt, W, b):
    return pl.pallas_call(
        _layer_body,
        grid=(N // BN,),
        in_specs=[
            pl.BlockSpec((2, BN, HH), lambda i: (0, i, 0)),
            pl.BlockSpec((2, BN, HH), lambda i: (0, i, 0)),
            pl.BlockSpec((2, BN, 8), lambda i: (0, i, 0)),
            pl.BlockSpec((2 * H, H), lambda i: (0, 0)),
            pl.BlockSpec((1, H), lambda i: (0, 0)),
        ],
        out_specs=pl.BlockSpec((2, BN, HH), lambda i: (0, i, 0)),
        out_shape=jax.ShapeDtypeStruct((2, N, HH), jnp.float32),
    )(h, summed, cnt, W, b.reshape(1, H))


def _pool_heads_body(h_ref, b_ref, wh_ref, bh_ref, o_ref, gs_ref, gc_ref):
    i = pl.program_id(0)

    @pl.when(i == 0)
    def _():
        gs_ref[...] = jnp.zeros_like(gs_ref)
        gc_ref[...] = jnp.zeros_like(gc_ref)

    bvals = b_ref[0, 0]
    gid = lax.broadcasted_iota(jnp.int32, (G, BN), 0)
    oh = (bvals[None, :] == gid).astype(jnp.float32)
    h_full = jnp.concatenate([h_ref[0], h_ref[1]], axis=1)
    gs_ref[...] += jnp.dot(oh, h_full, preferred_element_type=jnp.float32)
    gc_ref[...] += jnp.sum(oh, axis=1, keepdims=True)

    @pl.when(i == pl.num_programs(0) - 1)
    def _():
        hg = gs_ref[...] / jnp.maximum(gc_ref[...], 1.0)
        z = jnp.dot(hg, wh_ref[...], preferred_element_type=jnp.float32)
        z += bh_ref[...]
        qed = jax.nn.sigmoid(z[:, 1:2])
        o_ref[...] = jnp.concatenate([z[:, 0:1], qed, z[:, 2:4]], axis=1)


def _pool_heads(h, batch3d, W_heads, b_heads):
    return pl.pallas_call(
        _pool_heads_body,
        grid=(N // BN,),
        in_specs=[
            pl.BlockSpec((2, BN, HH), lambda i: (0, i, 0)),
            pl.BlockSpec((1, 1, BN), lambda i: (i, 0, 0)),
            pl.BlockSpec((H, 4), lambda i: (0, 0)),
            pl.BlockSpec((1, 4), lambda i: (0, 0)),
        ],
        out_specs=pl.BlockSpec((G, 4), lambda i: (0, 0)),
        out_shape=jax.ShapeDtypeStruct((G, 4), jnp.float32),
        scratch_shapes=[
            pltpu.VMEM((G, H), jnp.float32),
            pltpu.VMEM((G, 1), jnp.float32),
        ],
    )(h, batch3d, W_heads, b_heads.reshape(1, 4))


def kernel(x, edge_index, batch, W_in, b_in, W1, b1, W2, b2,
           W_aff, b_aff, W_qed, b_qed, W_sa, b_sa, W_tpsa, b_tpsa):
    src2d = edge_index[0].reshape(NROW, CH)
    dst2d = edge_index[1].reshape(NROW, CH)
    batch3d = batch.reshape(N // BN, 1, BN)
    z32 = jnp.zeros((N, HH), jnp.float32)
    z8 = jnp.zeros((N, 8), jnp.float32)
    ones_c = jnp.ones((CH, 8), jnp.float32)

    h = _input_proj(x, W_in, b_in)
    cnt = _make_sc_counts()(dst2d, z8, ones_c)

    agg = _make_sc_agg()
    s1 = agg(h, src2d, dst2d, z32)
    h = _layer_dense(h, s1, cnt, W1, b1)

    s2 = agg(h, src2d, dst2d, z32)
    h = _layer_dense(h, s2, cnt, W2, b2)

    W_heads = jnp.concatenate([W_aff, W_qed, W_sa, W_tpsa], axis=1)
    b_heads = jnp.concatenate([b_aff, b_qed, b_sa, b_tpsa])
    out = _pool_heads(h, batch3d, W_heads, b_heads)
    return (out[:, 0], out[:, 1], out[:, 2], out[:, 3])


# speedup vs baseline: 10.6470x; 1.1747x over previous
import functools

import jax
import jax.numpy as jnp
from jax import lax
from jax.experimental import pallas as pl
from jax.experimental.pallas import tpu as pltpu
from jax.experimental.pallas import tpu_sc as plsc

N = 50000
E = 800000
D_IN = 58
H = 64
HH = 32
G = 256

BN = 2000
CH = 256
NROW = E // CH
NTILE = 16
NBUF = 3
RPT = 3128

_BASE = NROW // NTILE
_EXTRA = NROW - _BASE * NTILE
_MCH = _BASE + 1


def _inproj_body(x_ref, w_ref, b_ref, o_ref):
    acc = jax.nn.relu(
        jnp.dot(x_ref[...], w_ref[...], preferred_element_type=jnp.float32)
        + b_ref[...]
    )
    o_ref[0] = acc[:, :HH]
    o_ref[1] = acc[:, HH:]


def _input_proj(x, W_in, b_in):
    return pl.pallas_call(
        _inproj_body,
        grid=(N // BN,),
        in_specs=[
            pl.BlockSpec((BN, D_IN), lambda i: (i, 0)),
            pl.BlockSpec((D_IN, H), lambda i: (0, 0)),
            pl.BlockSpec((1, H), lambda i: (0, 0)),
        ],
        out_specs=pl.BlockSpec((2, BN, HH), lambda i: (0, i, 0)),
        out_shape=jax.ShapeDtypeStruct((2, N, HH), jnp.float32),
    )(x, W_in, b_in.reshape(1, H))


def _per_tile_slice(s, fn):
    pl.when(s < NTILE - 1)(lambda: fn(s * RPT, RPT))
    pl.when(s == NTILE - 1)(lambda: fn((NTILE - 1) * RPT,
                                       N - (NTILE - 1) * RPT))


def _make_sc_agg():
    mesh = plsc.VectorSubcoreMesh(core_axis_name="c", subcore_axis_name="s")

    scratch = dict(
        sidx=pltpu.VMEM((NBUF, CH), jnp.int32),
        didx=pltpu.VMEM((NBUF, CH), jnp.int32),
        rows=pltpu.VMEM((NBUF, CH, HH), jnp.float32),
        accum=pltpu.VMEM_SHARED((N, HH), jnp.float32),
        isem_s=pltpu.SemaphoreType.DMA((NBUF,)),
        isem_d=pltpu.SemaphoreType.DMA((NBUF,)),
        gsem=pltpu.SemaphoreType.DMA((NBUF,)),
    )

    def body(h_hbm, src2d, dst2d, z32_hbm, s_out,
             sidx, didx, rows, accum, isem_s, isem_d, gsem):
        c = lax.axis_index("c")
        s = lax.axis_index("s")
        table = h_hbm.at[c]

        _per_tile_slice(s, lambda r0, ln: pltpu.sync_copy(
            z32_hbm.at[pl.ds(r0, ln)], accum.at[pl.ds(r0, ln)]))
        plsc.subcore_barrier()

        nch = _BASE + jnp.where(s < _EXTRA, 1, 0)
        start = s * _BASE + jnp.minimum(s, _EXTRA)

        def issue_idx(k):
            b = lax.rem(k, NBUF)
            row = start + k
            pltpu.async_copy(src2d.at[row], sidx.at[b], isem_s.at[b])
            pltpu.async_copy(dst2d.at[row], didx.at[b], isem_d.at[b])

        def issue_gather(k):
            b = lax.rem(k, NBUF)
            row = start + k
            pltpu.make_async_copy(src2d.at[row], sidx.at[b],
                                  isem_s.at[b]).wait()
            pltpu.async_copy(table.at[sidx.at[b]], rows.at[b], gsem.at[b])

        def consume(k):
            b = lax.rem(k, NBUF)
            row = start + k
            pltpu.make_async_copy(dst2d.at[row], didx.at[b],
                                  isem_d.at[b]).wait()
            pltpu.make_async_copy(table.at[sidx.at[b]], rows.at[b],
                                  gsem.at[b]).wait()
            pltpu.sync_copy(rows.at[b], accum.at[didx.at[b]], add=True)

        def step(k, _):
            pl.when(k < nch)(lambda: issue_idx(k))
            pl.when((k >= 1) & (k - 1 < nch))(lambda: issue_gather(k - 1))
            pl.when((k >= 2) & (k - 2 < nch))(lambda: consume(k - 2))
            return _

        lax.fori_loop(0, _MCH + 2, step, None)

        plsc.subcore_barrier()

        _per_tile_slice(s, lambda r0, ln: pltpu.sync_copy(
            accum.at[pl.ds(r0, ln)], s_out.at[c, pl.ds(r0, ln)]))

    return pl.kernel(
        body,
        out_type=jax.ShapeDtypeStruct((2, N, HH), jnp.float32),
        mesh=mesh,
        scratch_types=scratch,
        compiler_params=pltpu.CompilerParams(use_tc_tiling_on_sc=False),
    )


SLAB = 5
_NSLAB = NROW // SLAB
_NW = 2 * NTILE
_SBASE = _NSLAB // _NW
_SEXTRA = _NSLAB - _SBASE * _NW
_MSL = _SBASE + 1


def _make_sc_counts():
    mesh = plsc.VectorSubcoreMesh(core_axis_name="c", subcore_axis_name="s")

    scratch = dict(
        slab=pltpu.VMEM((NBUF, SLAB, CH), jnp.int32),
        ones_v=pltpu.VMEM((CH, 8), jnp.float32),
        cnt_sh=pltpu.VMEM_SHARED((N, 8), jnp.float32),
        dsem=pltpu.SemaphoreType.DMA((NBUF,)),
    )

    def body(dst2d, z8_hbm, ones_hbm, c_out,
             slab, ones_v, cnt_sh, dsem):
        c = lax.axis_index("c")
        s = lax.axis_index("s")
        w = c * NTILE + s

        _per_tile_slice(s, lambda r0, ln: pltpu.sync_copy(
            z8_hbm.at[pl.ds(r0, ln)], cnt_sh.at[pl.ds(r0, ln)]))
        pltpu.sync_copy(ones_hbm, ones_v)
        plsc.subcore_barrier()

        nsl = _SBASE + jnp.where(w < _SEXTRA, 1, 0)
        start = w * _SBASE + jnp.minimum(w, _SEXTRA)

        def issue(k):
            b = lax.rem(k, NBUF)
            row0 = (start + k) * SLAB
            pltpu.async_copy(dst2d.at[pl.ds(row0, SLAB)], slab.at[b],
                             dsem.at[b])

        def consume(k):
            b = lax.rem(k, NBUF)
            row0 = (start + k) * SLAB
            pltpu.make_async_copy(dst2d.at[pl.ds(row0, SLAB)], slab.at[b],
                                  dsem.at[b]).wait()
            for j in range(SLAB):
                pltpu.sync_copy(ones_v, cnt_sh.at[slab.at[b, j]], add=True)

        def step(k, _):
            pl.when(k < nsl)(lambda: issue(k))
            pl.when((k >= 1) & (k - 1 < nsl))(lambda: consume(k - 1))
            return _

        lax.fori_loop(0, _MSL + 1, step, None)

        plsc.subcore_barrier()

        _per_tile_slice(s, lambda r0, ln: pltpu.sync_copy(
            cnt_sh.at[pl.ds(r0, ln)], c_out.at[c, pl.ds(r0, ln)]))

    return pl.kernel(
        body,
        out_type=jax.ShapeDtypeStruct((2, N, 8), jnp.float32),
        mesh=mesh,
        scratch_types=scratch,
        compiler_params=pltpu.CompilerParams(use_tc_tiling_on_sc=False),
    )


def _layer_body(h_ref, s_ref, c_ref, w_ref, b_ref, o_ref):
    cnt = c_ref[0][:, 0:1] + c_ref[1][:, 0:1]
    inv = 1.0 / jnp.maximum(cnt, 1.0)
    h_full = jnp.concatenate([h_ref[0], h_ref[1]], axis=1)
    agg = jnp.concatenate([s_ref[0], s_ref[1]], axis=1) * inv
    acc = jnp.dot(h_full, w_ref[:H], preferred_element_type=jnp.float32)
    acc += jnp.dot(agg, w_ref[H:], preferred_element_type=jnp.float32)
    acc = jax.nn.relu(acc + b_ref[...])
    o_ref[0] = acc[:, :HH]
    o_ref[1] = acc[:, HH:]


def _layer_dense(h, summed, cnt, W, b):
    return pl.pallas_call(
        _layer_body,
        grid=(N // BN,),
        in_specs=[
            pl.BlockSpec((2, BN, HH), lambda i: (0, i, 0)),
            pl.BlockSpec((2, BN, HH), lambda i: (0, i, 0)),
            pl.BlockSpec((2, BN, 8), lambda i: (0, i, 0)),
            pl.BlockSpec((2 * H, H), lambda i: (0, 0)),
            pl.BlockSpec((1, H), lambda i: (0, 0)),
        ],
        out_specs=pl.BlockSpec((2, BN, HH), lambda i: (0, i, 0)),
        out_shape=jax.ShapeDtypeStruct((2, N, HH), jnp.float32),
    )(h, summed, cnt, W, b.reshape(1, H))


def _pool_heads_body(h_ref, b_ref, wh_ref, bh_ref, o_ref, gs_ref, gc_ref):
    i = pl.program_id(0)

    @pl.when(i == 0)
    def _():
        gs_ref[...] = jnp.zeros_like(gs_ref)
        gc_ref[...] = jnp.zeros_like(gc_ref)

    bvals = b_ref[0, 0]
    gid = lax.broadcasted_iota(jnp.int32, (G, BN), 0)
    oh = (bvals[None, :] == gid).astype(jnp.float32)
    h_full = jnp.concatenate([h_ref[0], h_ref[1]], axis=1)
    gs_ref[...] += jnp.dot(oh, h_full, preferred_element_type=jnp.float32)
    gc_ref[...] += jnp.sum(oh, axis=1, keepdims=True)

    @pl.when(i == pl.num_programs(0) - 1)
    def _():
        hg = gs_ref[...] / jnp.maximum(gc_ref[...], 1.0)
        z = jnp.dot(hg, wh_ref[...], preferred_element_type=jnp.float32)
        z += bh_ref[...]
        qed = jax.nn.sigmoid(z[:, 1:2])
        o_ref[...] = jnp.concatenate([z[:, 0:1], qed, z[:, 2:4]], axis=1)


def _pool_heads(h, batch3d, W_heads, b_heads):
    return pl.pallas_call(
        _pool_heads_body,
        grid=(N // BN,),
        in_specs=[
            pl.BlockSpec((2, BN, HH), lambda i: (0, i, 0)),
            pl.BlockSpec((1, 1, BN), lambda i: (i, 0, 0)),
            pl.BlockSpec((H, 4), lambda i: (0, 0)),
            pl.BlockSpec((1, 4), lambda i: (0, 0)),
        ],
        out_specs=pl.BlockSpec((G, 4), lambda i: (0, 0)),
        out_shape=jax.ShapeDtypeStruct((G, 4), jnp.float32),
        scratch_shapes=[
            pltpu.VMEM((G, H), jnp.float32),
            pltpu.VMEM((G, 1), jnp.float32),
        ],
    )(h, batch3d, W_heads, b_heads.reshape(1, 4))


def kernel(x, edge_index, batch, W_in, b_in, W1, b1, W2, b2,
           W_aff, b_aff, W_qed, b_qed, W_sa, b_sa, W_tpsa, b_tpsa):
    src2d = edge_index[0].reshape(NROW, CH)
    dst2d = edge_index[1].reshape(NROW, CH)
    batch3d = batch.reshape(N // BN, 1, BN)
    z32 = jnp.zeros((N, HH), jnp.float32)
    z8 = jnp.zeros((N, 8), jnp.float32)
    ones_c = jnp.ones((CH, 8), jnp.float32)

    h = _input_proj(x, W_in, b_in)
    cnt = _make_sc_counts()(dst2d, z8, ones_c)

    agg = _make_sc_agg()
    s1 = agg(h, src2d, dst2d, z32)
    h = _layer_dense(h, s1, cnt, W1, b1)

    s2 = agg(h, src2d, dst2d, z32)
    h = _layer_dense(h, s2, cnt, W2, b2)

    W_heads = jnp.concatenate([W_aff, W_qed, W_sa, W_tpsa], axis=1)
    b_heads = jnp.concatenate([b_aff, b_qed, b_sa, b_tpsa])
    out = _pool_heads(h, batch3d, W_heads, b_heads)
    return (out[:, 0], out[:, 1], out[:, 2], out[:, 3])


# speedup vs baseline: 14.3254x; 1.3455x over previous
import functools

import jax
import jax.numpy as jnp
from jax import lax
from jax.experimental import pallas as pl
from jax.experimental.pallas import tpu as pltpu
from jax.experimental.pallas import tpu_sc as plsc

N = 50000
E = 800000
D_IN = 58
H = 64
HH = 32
G = 256

BN = 2048
CH = 256
NROW = E // CH
NTILE = 16
NBUF = 3
RPT = 3128

_BASE = NROW // NTILE
_EXTRA = NROW - _BASE * NTILE
_MCH = _BASE + 1


BP = BN // 4
NP = N // 4


def _inproj_body(x_ref, w_ref, b_ref, o_ref):
    acc4 = jax.nn.relu(
        jnp.dot(x_ref[...], w_ref[...], preferred_element_type=jnp.float32)
        + b_ref[...]
    )
    o_ref[0] = jnp.concatenate(
        [acc4[:, 64 * j:64 * j + HH] for j in range(4)], axis=1)
    o_ref[1] = jnp.concatenate(
        [acc4[:, 64 * j + HH:64 * j + H] for j in range(4)], axis=1)


def _input_proj(x4, W4_in, b4_in):
    return pl.pallas_call(
        _inproj_body,
        grid=(pl.cdiv(NP, BP),),
        in_specs=[
            pl.BlockSpec((BP, 4 * D_IN), lambda i: (i, 0)),
            pl.BlockSpec((4 * D_IN, 4 * H), lambda i: (0, 0)),
            pl.BlockSpec((1, 4 * H), lambda i: (0, 0)),
        ],
        out_specs=pl.BlockSpec((2, BP, 128), lambda i: (0, i, 0)),
        out_shape=jax.ShapeDtypeStruct((2, NP, 128), jnp.float32),
    )(x4, W4_in, b4_in)


def _per_tile_slice(s, fn):
    pl.when(s < NTILE - 1)(lambda: fn(s * RPT, RPT))
    pl.when(s == NTILE - 1)(lambda: fn((NTILE - 1) * RPT,
                                       N - (NTILE - 1) * RPT))


def _make_sc_agg():
    mesh = plsc.VectorSubcoreMesh(core_axis_name="c", subcore_axis_name="s")

    scratch = dict(
        sidx=pltpu.VMEM((NBUF, CH), jnp.int32),
        didx=pltpu.VMEM((NBUF, CH), jnp.int32),
        rows=pltpu.VMEM((NBUF, CH, HH), jnp.float32),
        accum=pltpu.VMEM_SHARED((N, HH), jnp.float32),
        isem_s=pltpu.SemaphoreType.DMA((NBUF,)),
        isem_d=pltpu.SemaphoreType.DMA((NBUF,)),
        gsem=pltpu.SemaphoreType.DMA((NBUF,)),
    )

    def body(h_hbm, src2d, dst2d, z32_hbm, s_out,
             sidx, didx, rows, accum, isem_s, isem_d, gsem):
        c = lax.axis_index("c")
        s = lax.axis_index("s")
        table = h_hbm.at[c]

        _per_tile_slice(s, lambda r0, ln: pltpu.sync_copy(
            z32_hbm.at[pl.ds(r0, ln)], accum.at[pl.ds(r0, ln)]))
        plsc.subcore_barrier()

        nch = _BASE + jnp.where(s < _EXTRA, 1, 0)
        start = s * _BASE + jnp.minimum(s, _EXTRA)

        def issue_idx(k):
            b = lax.rem(k, NBUF)
            row = start + k
            pltpu.async_copy(src2d.at[row], sidx.at[b], isem_s.at[b])
            pltpu.async_copy(dst2d.at[row], didx.at[b], isem_d.at[b])

        def issue_gather(k):
            b = lax.rem(k, NBUF)
            row = start + k
            pltpu.make_async_copy(src2d.at[row], sidx.at[b],
                                  isem_s.at[b]).wait()
            pltpu.async_copy(table.at[sidx.at[b]], rows.at[b], gsem.at[b])

        def consume(k):
            b = lax.rem(k, NBUF)
            row = start + k
            pltpu.make_async_copy(dst2d.at[row], didx.at[b],
                                  isem_d.at[b]).wait()
            pltpu.make_async_copy(table.at[sidx.at[b]], rows.at[b],
                                  gsem.at[b]).wait()
            pltpu.sync_copy(rows.at[b], accum.at[didx.at[b]], add=True)

        def step(k, _):
            pl.when(k < nch)(lambda: issue_idx(k))
            pl.when((k >= 1) & (k - 1 < nch))(lambda: issue_gather(k - 1))
            pl.when((k >= 2) & (k - 2 < nch))(lambda: consume(k - 2))
            return _

        lax.fori_loop(0, _MCH + 2, step, None)

        plsc.subcore_barrier()

        _per_tile_slice(s, lambda r0, ln: pltpu.sync_copy(
            accum.at[pl.ds(r0, ln)], s_out.at[c, pl.ds(r0, ln)]))

    return pl.kernel(
        body,
        out_type=jax.ShapeDtypeStruct((2, N, HH), jnp.float32),
        mesh=mesh,
        scratch_types=scratch,
        compiler_params=pltpu.CompilerParams(use_tc_tiling_on_sc=False),
    )


SLAB = 5
_NSLAB = NROW // SLAB
_NW = 2 * NTILE
_SBASE = _NSLAB // _NW
_SEXTRA = _NSLAB - _SBASE * _NW
_MSL = _SBASE + 1


def _make_sc_counts():
    mesh = plsc.VectorSubcoreMesh(core_axis_name="c", subcore_axis_name="s")

    scratch = dict(
        slab=pltpu.VMEM((NBUF, SLAB, CH), jnp.int32),
        ones_v=pltpu.VMEM((CH, 8), jnp.float32),
        cnt_sh=pltpu.VMEM_SHARED((N, 8), jnp.float32),
        dsem=pltpu.SemaphoreType.DMA((NBUF,)),
    )

    def body(dst2d, z8_hbm, ones_hbm, c_out,
             slab, ones_v, cnt_sh, dsem):
        c = lax.axis_index("c")
        s = lax.axis_index("s")
        w = c * NTILE + s

        _per_tile_slice(s, lambda r0, ln: pltpu.sync_copy(
            z8_hbm.at[pl.ds(r0, ln)], cnt_sh.at[pl.ds(r0, ln)]))
        pltpu.sync_copy(ones_hbm, ones_v)
        plsc.subcore_barrier()

        nsl = _SBASE + jnp.where(w < _SEXTRA, 1, 0)
        start = w * _SBASE + jnp.minimum(w, _SEXTRA)

        def issue(k):
            b = lax.rem(k, NBUF)
            row0 = (start + k) * SLAB
            pltpu.async_copy(dst2d.at[pl.ds(row0, SLAB)], slab.at[b],
                             dsem.at[b])

        def consume(k):
            b = lax.rem(k, NBUF)
            row0 = (start + k) * SLAB
            pltpu.make_async_copy(dst2d.at[pl.ds(row0, SLAB)], slab.at[b],
                                  dsem.at[b]).wait()
            for j in range(SLAB):
                pltpu.sync_copy(ones_v, cnt_sh.at[slab.at[b, j]], add=True)

        def step(k, _):
            pl.when(k < nsl)(lambda: issue(k))
            pl.when((k >= 1) & (k - 1 < nsl))(lambda: consume(k - 1))
            return _

        lax.fori_loop(0, _MSL + 1, step, None)

        plsc.subcore_barrier()

        _per_tile_slice(s, lambda r0, ln: pltpu.sync_copy(
            cnt_sh.at[pl.ds(r0, ln)], c_out.at[c, pl.ds(r0, ln)]))

    return pl.kernel(
        body,
        out_type=jax.ShapeDtypeStruct((2, N, 8), jnp.float32),
        mesh=mesh,
        scratch_types=scratch,
        compiler_params=pltpu.CompilerParams(use_tc_tiling_on_sc=False),
    )


def _layer_body(h_ref, s_ref, c_ref, wa_ref, wb_ref, wc_ref, wd_ref,
                b_ref, o_ref):
    inv = 1.0 / jnp.maximum(c_ref[...], 1.0)
    s0 = s_ref[0] * inv
    s1 = s_ref[1] * inv
    acc4 = jnp.dot(h_ref[0], wa_ref[...], preferred_element_type=jnp.float32)
    acc4 += jnp.dot(h_ref[1], wb_ref[...], preferred_element_type=jnp.float32)
    acc4 += jnp.dot(s0, wc_ref[...], preferred_element_type=jnp.float32)
    acc4 += jnp.dot(s1, wd_ref[...], preferred_element_type=jnp.float32)
    acc4 = jax.nn.relu(acc4 + b_ref[...])
    o_ref[0] = jnp.concatenate(
        [acc4[:, 64 * j:64 * j + HH] for j in range(4)], axis=1)
    o_ref[1] = jnp.concatenate(
        [acc4[:, 64 * j + HH:64 * j + H] for j in range(4)], axis=1)


def _layer_dense(h, summed, c32p, W4s, b4):
    return pl.pallas_call(
        _layer_body,
        grid=(pl.cdiv(NP, BP),),
        in_specs=[
            pl.BlockSpec((2, BP, 128), lambda i: (0, i, 0)),
            pl.BlockSpec((2, BP, 128), lambda i: (0, i, 0)),
            pl.BlockSpec((BP, 128), lambda i: (i, 0)),
            pl.BlockSpec((128, 4 * H), lambda i: (0, 0)),
            pl.BlockSpec((128, 4 * H), lambda i: (0, 0)),
            pl.BlockSpec((128, 4 * H), lambda i: (0, 0)),
            pl.BlockSpec((128, 4 * H), lambda i: (0, 0)),
            pl.BlockSpec((1, 4 * H), lambda i: (0, 0)),
        ],
        out_specs=pl.BlockSpec((2, BP, 128), lambda i: (0, i, 0)),
        out_shape=jax.ShapeDtypeStruct((2, NP, 128), jnp.float32),
    )(h, summed, c32p, *W4s, b4)


def _pool_heads_body(h_ref, b0_ref, b1_ref, b2_ref, b3_ref,
                     wh_ref, bh_ref, o_ref, gs_ref, gc_ref):
    i = pl.program_id(0)

    @pl.when(i == 0)
    def _():
        gs_ref[...] = jnp.zeros_like(gs_ref)
        gc_ref[...] = jnp.zeros_like(gc_ref)

    valid = lax.broadcasted_iota(jnp.int32, (BP, 1), 0) + i * BP < NP
    h0 = jnp.where(valid, h_ref[0], 0.0)
    h1 = jnp.where(valid, h_ref[1], 0.0)
    gid = lax.broadcasted_iota(jnp.int32, (G, BP), 0)
    for j, bj_ref in enumerate((b0_ref, b1_ref, b2_ref, b3_ref)):
        bvals = bj_ref[0, 0]
        oh = (bvals[None, :] == gid).astype(jnp.float32)
        hj = jnp.concatenate([h0[:, HH * j:HH * j + HH],
                              h1[:, HH * j:HH * j + HH]], axis=1)
        gs_ref[...] += jnp.dot(oh, hj, preferred_element_type=jnp.float32)
        gc_ref[...] += jnp.sum(oh, axis=1, keepdims=True)

    @pl.when(i == pl.num_programs(0) - 1)
    def _():
        hg = gs_ref[...] / jnp.maximum(gc_ref[...], 1.0)
        z = jnp.dot(hg, wh_ref[...], preferred_element_type=jnp.float32)
        z += bh_ref[...]
        qed = jax.nn.sigmoid(z[:, 1:2])
        o_ref[...] = jnp.concatenate([z[:, 0:1], qed, z[:, 2:4]], axis=1)


def _pool_heads(h, batch_js, W_heads, b_heads):
    nblk = pl.cdiv(NP, BP)
    return pl.pallas_call(
        _pool_heads_body,
        grid=(nblk,),
        in_specs=[
            pl.BlockSpec((2, BP, 128), lambda i: (0, i, 0)),
            pl.BlockSpec((1, 1, BP), lambda i: (i, 0, 0)),
            pl.BlockSpec((1, 1, BP), lambda i: (i, 0, 0)),
            pl.BlockSpec((1, 1, BP), lambda i: (i, 0, 0)),
            pl.BlockSpec((1, 1, BP), lambda i: (i, 0, 0)),
            pl.BlockSpec((H, 4), lambda i: (0, 0)),
            pl.BlockSpec((1, 4), lambda i: (0, 0)),
        ],
        out_specs=pl.BlockSpec((G, 4), lambda i: (0, 0)),
        out_shape=jax.ShapeDtypeStruct((G, 4), jnp.float32),
        scratch_shapes=[
            pltpu.VMEM((G, H), jnp.float32),
            pltpu.VMEM((G, 1), jnp.float32),
        ],
    )(h, *batch_js, W_heads, b_heads.reshape(1, 4))


def kernel(x, edge_index, batch, W_in, b_in, W1, b1, W2, b2,
           W_aff, b_aff, W_qed, b_qed, W_sa, b_sa, W_tpsa, b_tpsa):
    src2d = edge_index[0].reshape(NROW, CH)
    dst2d = edge_index[1].reshape(NROW, CH)
    z32 = jnp.zeros((N, HH), jnp.float32)
    z8 = jnp.zeros((N, 8), jnp.float32)
    ones_c = jnp.ones((CH, 8), jnp.float32)

    eye4 = jnp.eye(4, dtype=jnp.float32)
    x4 = x.reshape(NP, 4 * D_IN)
    W4_in = jnp.kron(eye4, W_in)
    b4_in = jnp.tile(b_in, 4).reshape(1, 4 * H)

    def w4s(W):
        return tuple(jnp.kron(eye4, W[HH * j:HH * (j + 1)]) for j in range(4))

    nblk = pl.cdiv(NP, BP)
    batch_pad = jnp.pad(batch, (0, nblk * BP * 4 - N), constant_values=G)
    batch_js = [batch_pad[j::4].reshape(nblk, 1, BP) for j in range(4)]

    h = _input_proj(x4, W4_in, b4_in)
    cnt = _make_sc_counts()(dst2d, z8, ones_c)
    c = cnt[0, :, 0] + cnt[1, :, 0]
    c32p = jnp.broadcast_to(c[:, None], (N, HH)).reshape(NP, 128)

    agg = _make_sc_agg()
    s1 = agg(h.reshape(2, N, HH), src2d, dst2d, z32)
    h = _layer_dense(h, s1.reshape(2, NP, 128), c32p, w4s(W1),
                     jnp.tile(b1, 4).reshape(1, 4 * H))

    s2 = agg(h.reshape(2, N, HH), src2d, dst2d, z32)
    h = _layer_dense(h, s2.reshape(2, NP, 128), c32p, w4s(W2),
                     jnp.tile(b2, 4).reshape(1, 4 * H))

    W_heads = jnp.concatenate([W_aff, W_qed, W_sa, W_tpsa], axis=1)
    b_heads = jnp.concatenate([b_aff, b_qed, b_sa, b_tpsa])
    out = _pool_heads(h, batch_js, W_heads, b_heads)
    return (out[:, 0], out[:, 1], out[:, 2], out[:, 3])


# speedup vs baseline: 15.3510x; 1.0716x over previous
import functools

import jax
import jax.numpy as jnp
from jax import lax
from jax.experimental import pallas as pl
from jax.experimental.pallas import tpu as pltpu
from jax.experimental.pallas import tpu_sc as plsc

N = 50000
E = 800000
D_IN = 58
H = 64
HH = 32
G = 256

BN = 2048
CH = 256
NROW = E // CH
NTILE = 16
NBUF = 3
RPT = 3128

_BASE = NROW // NTILE
_EXTRA = NROW - _BASE * NTILE
_MCH = _BASE + 1


BP = BN // 4
NP = N // 4


def _inproj_body(x_ref, w_ref, b_ref, o_ref):
    acc4 = jax.nn.relu(
        jnp.dot(x_ref[...], w_ref[...], preferred_element_type=jnp.float32)
        + b_ref[...]
    )
    o_ref[0] = jnp.concatenate(
        [acc4[:, 64 * j:64 * j + HH] for j in range(4)], axis=1)
    o_ref[1] = jnp.concatenate(
        [acc4[:, 64 * j + HH:64 * j + H] for j in range(4)], axis=1)


def _input_proj(x4, W4_in, b4_in):
    return pl.pallas_call(
        _inproj_body,
        grid=(pl.cdiv(NP, BP),),
        in_specs=[
            pl.BlockSpec((BP, 4 * D_IN), lambda i: (i, 0)),
            pl.BlockSpec((4 * D_IN, 4 * H), lambda i: (0, 0)),
            pl.BlockSpec((1, 4 * H), lambda i: (0, 0)),
        ],
        out_specs=pl.BlockSpec((2, BP, 128), lambda i: (0, i, 0)),
        out_shape=jax.ShapeDtypeStruct((2, NP, 128), jnp.float32),
    )(x4, W4_in, b4_in)


def _per_tile_slice(s, fn):
    pl.when(s < NTILE - 1)(lambda: fn(s * RPT, RPT))
    pl.when(s == NTILE - 1)(lambda: fn((NTILE - 1) * RPT,
                                       N - (NTILE - 1) * RPT))


def _make_sc_agg():
    mesh = plsc.VectorSubcoreMesh(core_axis_name="c", subcore_axis_name="s")

    scratch = dict(
        sidx=pltpu.VMEM((NBUF, CH), jnp.int32),
        didx=pltpu.VMEM((NBUF, CH), jnp.int32),
        rows=pltpu.VMEM((NBUF, CH, HH), jnp.float32),
        accum=pltpu.VMEM_SHARED((N, HH), jnp.float32),
        isem_s=pltpu.SemaphoreType.DMA((NBUF,)),
        isem_d=pltpu.SemaphoreType.DMA((NBUF,)),
        gsem=pltpu.SemaphoreType.DMA((NBUF,)),
        ssem=pltpu.SemaphoreType.DMA((NBUF,)),
    )

    def body(h_hbm, src2d, dst2d, z32_hbm, s_out,
             sidx, didx, rows, accum, isem_s, isem_d, gsem, ssem):
        c = lax.axis_index("c")
        s = lax.axis_index("s")
        table = h_hbm.at[c]

        _per_tile_slice(s, lambda r0, ln: pltpu.sync_copy(
            z32_hbm.at[pl.ds(r0, ln)], accum.at[pl.ds(r0, ln)]))
        plsc.subcore_barrier()

        nch = _BASE + jnp.where(s < _EXTRA, 1, 0)
        start = s * _BASE + jnp.minimum(s, _EXTRA)

        def issue_idx(k):
            b = lax.rem(k, NBUF)
            row = start + k
            pltpu.async_copy(src2d.at[row], sidx.at[b], isem_s.at[b])
            pltpu.async_copy(dst2d.at[row], didx.at[b], isem_d.at[b])

        def issue_gather(k):
            b = lax.rem(k, NBUF)
            row = start + k
            pltpu.make_async_copy(src2d.at[row], sidx.at[b],
                                  isem_s.at[b]).wait()
            pltpu.async_copy(table.at[sidx.at[b]], rows.at[b], gsem.at[b])

        def issue_scatter(k):
            b = lax.rem(k, NBUF)
            row = start + k
            pltpu.make_async_copy(dst2d.at[row], didx.at[b],
                                  isem_d.at[b]).wait()
            pltpu.make_async_copy(table.at[sidx.at[b]], rows.at[b],
                                  gsem.at[b]).wait()
            pltpu.async_copy(rows.at[b], accum.at[didx.at[b]], ssem.at[b],
                             add=True)

        def drain_scatter(k):
            b = lax.rem(k, NBUF)
            pltpu.make_async_copy(rows.at[b], accum.at[didx.at[b]],
                                  ssem.at[b]).wait()

        def step(k, _):
            pl.when(k < nch)(lambda: issue_idx(k))
            pl.when((k >= 1) & (k - 1 < nch))(lambda: issue_gather(k - 1))
            pl.when((k >= 2) & (k - 2 < nch))(lambda: issue_scatter(k - 2))
            pl.when((k >= 3) & (k - 3 < nch))(lambda: drain_scatter(k - 3))
            return _

        lax.fori_loop(0, _MCH + 3, step, None)

        plsc.subcore_barrier()

        _per_tile_slice(s, lambda r0, ln: pltpu.sync_copy(
            accum.at[pl.ds(r0, ln)], s_out.at[c, pl.ds(r0, ln)]))

    return pl.kernel(
        body,
        out_type=jax.ShapeDtypeStruct((2, N, HH), jnp.float32),
        mesh=mesh,
        scratch_types=scratch,
        compiler_params=pltpu.CompilerParams(use_tc_tiling_on_sc=False),
    )


SLAB = 5
_NSLAB = NROW // SLAB
_NW = 2 * NTILE
_SBASE = _NSLAB // _NW
_SEXTRA = _NSLAB - _SBASE * _NW
_MSL = _SBASE + 1


def _make_sc_counts():
    mesh = plsc.VectorSubcoreMesh(core_axis_name="c", subcore_axis_name="s")

    scratch = dict(
        slab=pltpu.VMEM((NBUF, SLAB, CH), jnp.int32),
        ones_v=pltpu.VMEM((CH, 8), jnp.float32),
        cnt_sh=pltpu.VMEM_SHARED((N, 8), jnp.float32),
        dsem=pltpu.SemaphoreType.DMA((NBUF,)),
    )

    def body(dst2d, z8_hbm, ones_hbm, c_out,
             slab, ones_v, cnt_sh, dsem):
        c = lax.axis_index("c")
        s = lax.axis_index("s")
        w = c * NTILE + s

        _per_tile_slice(s, lambda r0, ln: pltpu.sync_copy(
            z8_hbm.at[pl.ds(r0, ln)], cnt_sh.at[pl.ds(r0, ln)]))
        pltpu.sync_copy(ones_hbm, ones_v)
        plsc.subcore_barrier()

        nsl = _SBASE + jnp.where(w < _SEXTRA, 1, 0)
        start = w * _SBASE + jnp.minimum(w, _SEXTRA)

        def issue(k):
            b = lax.rem(k, NBUF)
            row0 = (start + k) * SLAB
            pltpu.async_copy(dst2d.at[pl.ds(row0, SLAB)], slab.at[b],
                             dsem.at[b])

        def consume(k):
            b = lax.rem(k, NBUF)
            row0 = (start + k) * SLAB
            pltpu.make_async_copy(dst2d.at[pl.ds(row0, SLAB)], slab.at[b],
                                  dsem.at[b]).wait()
            for j in range(SLAB):
                pltpu.sync_copy(ones_v, cnt_sh.at[slab.at[b, j]], add=True)

        def step(k, _):
            pl.when(k < nsl)(lambda: issue(k))
            pl.when((k >= 1) & (k - 1 < nsl))(lambda: consume(k - 1))
            return _

        lax.fori_loop(0, _MSL + 1, step, None)

        plsc.subcore_barrier()

        _per_tile_slice(s, lambda r0, ln: pltpu.sync_copy(
            cnt_sh.at[pl.ds(r0, ln)], c_out.at[c, pl.ds(r0, ln)]))

    return pl.kernel(
        body,
        out_type=jax.ShapeDtypeStruct((2, N, 8), jnp.float32),
        mesh=mesh,
        scratch_types=scratch,
        compiler_params=pltpu.CompilerParams(use_tc_tiling_on_sc=False),
    )


def _layer_body(h_ref, s_ref, c_ref, wa_ref, wb_ref, wc_ref, wd_ref,
                b_ref, o_ref):
    inv = 1.0 / jnp.maximum(c_ref[...], 1.0)
    s0 = s_ref[0] * inv
    s1 = s_ref[1] * inv
    acc4 = jnp.dot(h_ref[0], wa_ref[...], preferred_element_type=jnp.float32)
    acc4 += jnp.dot(h_ref[1], wb_ref[...], preferred_element_type=jnp.float32)
    acc4 += jnp.dot(s0, wc_ref[...], preferred_element_type=jnp.float32)
    acc4 += jnp.dot(s1, wd_ref[...], preferred_element_type=jnp.float32)
    acc4 = jax.nn.relu(acc4 + b_ref[...])
    o_ref[0] = jnp.concatenate(
        [acc4[:, 64 * j:64 * j + HH] for j in range(4)], axis=1)
    o_ref[1] = jnp.concatenate(
        [acc4[:, 64 * j + HH:64 * j + H] for j in range(4)], axis=1)


def _layer_dense(h, summed, c32p, W4s, b4):
    return pl.pallas_call(
        _layer_body,
        grid=(pl.cdiv(NP, BP),),
        in_specs=[
            pl.BlockSpec((2, BP, 128), lambda i: (0, i, 0)),
            pl.BlockSpec((2, BP, 128), lambda i: (0, i, 0)),
            pl.BlockSpec((BP, 128), lambda i: (i, 0)),
            pl.BlockSpec((128, 4 * H), lambda i: (0, 0)),
            pl.BlockSpec((128, 4 * H), lambda i: (0, 0)),
            pl.BlockSpec((128, 4 * H), lambda i: (0, 0)),
            pl.BlockSpec((128, 4 * H), lambda i: (0, 0)),
            pl.BlockSpec((1, 4 * H), lambda i: (0, 0)),
        ],
        out_specs=pl.BlockSpec((2, BP, 128), lambda i: (0, i, 0)),
        out_shape=jax.ShapeDtypeStruct((2, NP, 128), jnp.float32),
    )(h, summed, c32p, *W4s, b4)


def _pool_heads_body(h_ref, b0_ref, b1_ref, b2_ref, b3_ref,
                     wh_ref, bh_ref, o_ref, gs_ref, gc_ref):
    i = pl.program_id(0)

    @pl.when(i == 0)
    def _():
        gs_ref[...] = jnp.zeros_like(gs_ref)
        gc_ref[...] = jnp.zeros_like(gc_ref)

    valid = lax.broadcasted_iota(jnp.int32, (BP, 1), 0) + i * BP < NP
    h0 = jnp.where(valid, h_ref[0], 0.0)
    h1 = jnp.where(valid, h_ref[1], 0.0)
    gid = lax.broadcasted_iota(jnp.int32, (G, BP), 0)
    for j, bj_ref in enumerate((b0_ref, b1_ref, b2_ref, b3_ref)):
        bvals = bj_ref[0, 0]
        oh = (bvals[None, :] == gid).astype(jnp.float32)
        hj = jnp.concatenate([h0[:, HH * j:HH * j + HH],
                              h1[:, HH * j:HH * j + HH]], axis=1)
        gs_ref[...] += jnp.dot(oh, hj, preferred_element_type=jnp.float32)
        gc_ref[...] += jnp.sum(oh, axis=1, keepdims=True)

    @pl.when(i == pl.num_programs(0) - 1)
    def _():
        hg = gs_ref[...] / jnp.maximum(gc_ref[...], 1.0)
        z = jnp.dot(hg, wh_ref[...], preferred_element_type=jnp.float32)
        z += bh_ref[...]
        qed = jax.nn.sigmoid(z[:, 1:2])
        o_ref[...] = jnp.concatenate([z[:, 0:1], qed, z[:, 2:4]], axis=1)


def _pool_heads(h, batch_js, W_heads, b_heads):
    nblk = pl.cdiv(NP, BP)
    return pl.pallas_call(
        _pool_heads_body,
        grid=(nblk,),
        in_specs=[
            pl.BlockSpec((2, BP, 128), lambda i: (0, i, 0)),
            pl.BlockSpec((1, 1, BP), lambda i: (i, 0, 0)),
            pl.BlockSpec((1, 1, BP), lambda i: (i, 0, 0)),
            pl.BlockSpec((1, 1, BP), lambda i: (i, 0, 0)),
            pl.BlockSpec((1, 1, BP), lambda i: (i, 0, 0)),
            pl.BlockSpec((H, 4), lambda i: (0, 0)),
            pl.BlockSpec((1, 4), lambda i: (0, 0)),
        ],
        out_specs=pl.BlockSpec((G, 4), lambda i: (0, 0)),
        out_shape=jax.ShapeDtypeStruct((G, 4), jnp.float32),
        scratch_shapes=[
            pltpu.VMEM((G, H), jnp.float32),
            pltpu.VMEM((G, 1), jnp.float32),
        ],
    )(h, *batch_js, W_heads, b_heads.reshape(1, 4))


def kernel(x, edge_index, batch, W_in, b_in, W1, b1, W2, b2,
           W_aff, b_aff, W_qed, b_qed, W_sa, b_sa, W_tpsa, b_tpsa):
    src2d = edge_index[0].reshape(NROW, CH)
    dst2d = edge_index[1].reshape(NROW, CH)
    z32 = jnp.zeros((N, HH), jnp.float32)
    z8 = jnp.zeros((N, 8), jnp.float32)
    ones_c = jnp.ones((CH, 8), jnp.float32)

    eye4 = jnp.eye(4, dtype=jnp.float32)
    x4 = x.reshape(NP, 4 * D_IN)
    W4_in = jnp.kron(eye4, W_in)
    b4_in = jnp.tile(b_in, 4).reshape(1, 4 * H)

    def w4s(W):
        return tuple(jnp.kron(eye4, W[HH * j:HH * (j + 1)]) for j in range(4))

    nblk = pl.cdiv(NP, BP)
    batch_pad = jnp.pad(batch, (0, nblk * BP * 4 - N), constant_values=G)
    batch_js = [batch_pad[j::4].reshape(nblk, 1, BP) for j in range(4)]

    h = _input_proj(x4, W4_in, b4_in)
    cnt = _make_sc_counts()(dst2d, z8, ones_c)
    c = cnt[0, :, 0] + cnt[1, :, 0]
    c32p = jnp.broadcast_to(c[:, None], (N, HH)).reshape(NP, 128)

    agg = _make_sc_agg()
    s1 = agg(h.reshape(2, N, HH), src2d, dst2d, z32)
    h = _layer_dense(h, s1.reshape(2, NP, 128), c32p, w4s(W1),
                     jnp.tile(b1, 4).reshape(1, 4 * H))

    s2 = agg(h.reshape(2, N, HH), src2d, dst2d, z32)
    h = _layer_dense(h, s2.reshape(2, NP, 128), c32p, w4s(W2),
                     jnp.tile(b2, 4).reshape(1, 4 * H))

    W_heads = jnp.concatenate([W_aff, W_qed, W_sa, W_tpsa], axis=1)
    b_heads = jnp.concatenate([b_aff, b_qed, b_sa, b_tpsa])
    out = _pool_heads(h, batch_js, W_heads, b_heads)
    return (out[:, 0], out[:, 1], out[:, 2], out[:, 3])
